# Initial kernel scaffold; baseline (speedup 1.0000x reference)
import jax
import jax.numpy as jnp
from jax.experimental import pallas as pl


def kernel(op_feats, device_feats, tensor_feats, link_feats, place_feats, prev_edge_index, link_edge_index, place_edge_index, op_W, op_b, dev_W, dev_b, et_W, et_b, gconv_W, gconv_b, fp_W, fp_b, fn_W, fn_b):
    raise NotImplementedError("write your pallas kernel here")


# speedup vs baseline: 1.3808x; 1.3808x over previous
import functools
import jax
import jax.numpy as jnp
from jax import lax
from jax.experimental import pallas as pl
from jax.experimental.pallas import tpu as pltpu

_NOP = 50000
_NDEV = 1024
_H = 64
_EH = 8
_NL = 6


def _elu(x):
    return jnp.where(x > 0, x, jnp.exp(jnp.minimum(x, 0.0)) - 1.0)


def _init_nodes_body(x_ref, w_ref, b_ref, o_ref):
    y = jnp.dot(x_ref[...], w_ref[0], preferred_element_type=jnp.float32)
    o_ref[0] = _elu(y + b_ref[0])


def _init_nodes(x, w, b, bm):
    n, din = x.shape
    nb = n // bm
    wst = w.reshape(din, 2, 32).transpose(1, 0, 2)
    bst = b.reshape(2, 1, 32)
    return pl.pallas_call(
        _init_nodes_body,
        grid=(2, nb),
        in_specs=[
            pl.BlockSpec((bm, din), lambda h, i: (i, 0)),
            pl.BlockSpec((1, din, 32), lambda h, i: (h, 0, 0)),
            pl.BlockSpec((1, 1, 32), lambda h, i: (h, 0, 0)),
        ],
        out_specs=pl.BlockSpec((1, bm, 32), lambda h, i: (h, i, 0)),
        out_shape=jax.ShapeDtypeStruct((2, n, 32), jnp.float32),
    )(x, wst, bst)


def _edge_mlp_body(x_ref, w_ref, b_ref, o_ref):
    z = jnp.dot(x_ref[...], w_ref[...], preferred_element_type=jnp.float32)
    z = z + b_ref[0]
    col = lax.broadcasted_iota(jnp.int32, z.shape, 1)
    o_ref[...] = jnp.where(col < _EH, _elu(z),
                           jnp.where(col == _EH, 1.0, 0.0))


def _edge_mlp(raw_pad, w, b, bm):
    ep = raw_pad.shape[0]
    w16 = jnp.pad(w, ((0, 0), (0, 16 - _EH)))
    b16 = jnp.pad(b, (0, 16 - _EH)).reshape(1, 16)
    return pl.pallas_call(
        _edge_mlp_body,
        grid=(ep // bm,),
        in_specs=[
            pl.BlockSpec((bm, 16), lambda i: (i, 0)),
            pl.BlockSpec((16, 16), lambda i: (0, 0)),
            pl.BlockSpec((1, 16), lambda i: (0, 0)),
        ],
        out_specs=pl.BlockSpec((bm, 16), lambda i: (i, 0)),
        out_shape=jax.ShapeDtypeStruct((ep, 16), jnp.float32),
    )(raw_pad, w16, b16)


def _combine_body(nark, op_ref,
                  g0l, g0h, g1l, g1h, g2l, g2h,
                  e0a, e0b, e1a, e1b, e2a, e2b,
                  w0, w1, w2, we0, we1, we2, o_ref,
                  *, last):
    gl = (g0l, g1l, g2l)
    gh = (g0h, g1h, g2h)
    ea = (e0a, e1a, e2a)
    eb = (e0b, e1b, e2b)
    ws = (w0, w1, w2)
    wes = (we0, we1, we2)
    acc = jnp.zeros(op_ref.shape[1:], jnp.float32)
    for k in range(nark):
        es = ea[k][0] + eb[k][0]
        cnt = es[:, _EH:_EH + 1]
        num = jnp.dot(gl[k][0], ws[k][0, 0], preferred_element_type=jnp.float32)
        num += jnp.dot(gh[k][0], ws[k][0, 1], preferred_element_type=jnp.float32)
        num += jnp.dot(es, wes[k][0], preferred_element_type=jnp.float32)
        acc += num / jnp.maximum(cnt, 1.0)
    o = op_ref[0] + acc * (1.0 / nark)
    o_ref[0] = o if last else _elu(o)


def _combine(opf, gs, ess, whs, wes, bs, last, bm):
    nark = len(gs)
    n = opf.shape[1]
    nb = n // bm
    wsts, wests = [], []
    for k in range(nark):
        w = whs[k].reshape(2, 32, 2, 32).transpose(2, 0, 1, 3)
        wsts.append(w)
        wep = jnp.zeros((16, 64), jnp.float32)
        wep = wep.at[:_EH].set(wes[k])
        wep = wep.at[_EH].set(bs[k])
        wests.append(wep.reshape(16, 2, 32).transpose(1, 0, 2))
    big = pl.BlockSpec((1, bm, 32), lambda h, i: (0, i, 0))
    bigh = pl.BlockSpec((1, bm, 32), lambda h, i: (1, i, 0))
    esa = pl.BlockSpec((1, bm, 16), lambda h, i: (0, i, 0))
    esb = pl.BlockSpec((1, bm, 16), lambda h, i: (1, i, 0))
    wsp = pl.BlockSpec((1, 2, 32, 32), lambda h, i: (h, 0, 0, 0))
    wesp = pl.BlockSpec((1, 16, 32), lambda h, i: (h, 0, 0))
    g3 = [gs[k] if k < nark else gs[0] for k in range(3)]
    e3 = [ess[k] if k < nark else ess[0] for k in range(3)]
    w3 = [wsts[k] if k < nark else wsts[0] for k in range(3)]
    we3 = [wests[k] if k < nark else wests[0] for k in range(3)]
    in_specs = [pl.BlockSpec((1, bm, 32), lambda h, i: (h, i, 0))]
    args = [opf]
    for k in range(3):
        in_specs += [big, bigh]
        args += [g3[k], g3[k]]
    for k in range(3):
        in_specs += [esa, esb]
        args += [e3[k], e3[k]]
    in_specs += [wsp] * 3 + [wesp] * 3
    args += w3 + we3
    return pl.pallas_call(
        functools.partial(_combine_body, len(gs), last=last),
        grid=(2, nb),
        in_specs=in_specs,
        out_specs=pl.BlockSpec((1, bm, 32), lambda h, i: (h, i, 0)),
        out_shape=jax.ShapeDtypeStruct((2, n, 32), jnp.float32),
    )(*args)


def _proj_head_body(xl_ref, xh_ref, w_ref, b_ref, o_ref):
    y = jnp.dot(xl_ref[0], w_ref[:32], preferred_element_type=jnp.float32)
    y += jnp.dot(xh_ref[0], w_ref[32:], preferred_element_type=jnp.float32)
    o_ref[...] = y + b_ref[...]


def _proj_head(opf, w64x16, b16, bm):
    n = opf.shape[1]
    return pl.pallas_call(
        _proj_head_body,
        grid=(n // bm,),
        in_specs=[
            pl.BlockSpec((1, bm, 32), lambda i: (0, i, 0)),
            pl.BlockSpec((1, bm, 32), lambda i: (1, i, 0)),
            pl.BlockSpec((64, 16), lambda i: (0, 0)),
            pl.BlockSpec((1, 16), lambda i: (0, 0)),
        ],
        out_specs=pl.BlockSpec((bm, 16), lambda i: (i, 0)),
        out_shape=jax.ShapeDtypeStruct((n, 16), jnp.float32),
    )(opf, opf, w64x16, b16.reshape(1, 16))


def _final_edge_body(ga_ref, gb_ref, ef_ref, w_ref, o_ref):
    y = ga_ref[...] + gb_ref[...]
    y += jnp.dot(ef_ref[...], w_ref[...], preferred_element_type=jnp.float32)
    o_ref[...] = y


def _final_edge(ga, gb, ef3, wfin, bm):
    ep = ga.shape[0]
    return pl.pallas_call(
        _final_edge_body,
        grid=(ep // bm,),
        in_specs=[
            pl.BlockSpec((bm, 16), lambda i: (i, 0)),
            pl.BlockSpec((bm, 16), lambda i: (i, 0)),
            pl.BlockSpec((bm, 16), lambda i: (i, 0)),
            pl.BlockSpec((16, 16), lambda i: (0, 0)),
        ],
        out_specs=pl.BlockSpec((bm, 16), lambda i: (i, 0)),
        out_shape=jax.ShapeDtypeStruct((ep, 16), jnp.float32),
    )(ga, gb, ef3, wfin)


def _segsum_nodes(opf, src, dst, n):
    x = jnp.concatenate([opf[0], opf[1]], axis=1)
    g = jax.ops.segment_sum(x[src], dst, num_segments=n)
    return jnp.stack([g[:, :32], g[:, 32:]])


def _segsum_ef(ef, dst, n):
    g = jax.ops.segment_sum(ef, dst, num_segments=n)
    return jnp.stack([g, jnp.zeros_like(g)])


def _gather_rows(tab, idx):
    return tab[idx]


def kernel(op_feats, device_feats, tensor_feats, link_feats, place_feats,
           prev_edge_index, link_edge_index, place_edge_index,
           op_W, op_b, dev_W, dev_b, et_W, et_b, gconv_W, gconv_b,
           fp_W, fp_b, fn_W, fn_b):
    e_t = tensor_feats.shape[0]
    e_p = place_feats.shape[0]
    e_l = link_feats.shape[0]

    def pad_rows(x, m=4096):
        r = (-x.shape[0]) % m
        return x if r == 0 else jnp.pad(x, ((0, r), (0, 0)))

    opf = _init_nodes(op_feats, op_W, op_b, bm=2000)
    devf = _init_nodes(device_feats, dev_W, dev_b, bm=1024)

    ef = [None] * 5
    ef[0] = _edge_mlp(pad_rows(link_feats), et_W[0], et_b[0], bm=4096)
    ef[1] = _edge_mlp(pad_rows(tensor_feats), et_W[1], et_b[1], bm=4096)
    ef[2] = _edge_mlp(pad_rows(tensor_feats), et_W[2], et_b[2], bm=4096)
    ef[3] = _edge_mlp(pad_rows(place_feats), et_W[3], et_b[3], bm=4096)
    ef[4] = _edge_mlp(pad_rows(place_feats), et_W[4], et_b[4], bm=4096)

    src_idx = [link_edge_index[0], prev_edge_index[0], prev_edge_index[1],
               place_edge_index[0], place_edge_index[1]]
    dst_idx = [link_edge_index[1], prev_edge_index[1], prev_edge_index[0],
               place_edge_index[1], place_edge_index[0]]
    n_edges = [e_l, e_t, e_t, e_p, e_p]
    src_is_op = [False, True, True, True, False]
    dst_is_op = [False, True, True, False, True]

    es = []
    for i in range(5):
        n = _NOP if dst_is_op[i] else _NDEV
        es.append(_segsum_ef(ef[i][:n_edges[i]], dst_idx[i], n))

    for l in range(_NL):
        gs = []
        for i in range(5):
            n = _NOP if dst_is_op[i] else _NDEV
            srcf = opf if src_is_op[i] else devf
            gs.append(_segsum_nodes(srcf, src_idx[i], dst_idx[i], n))
        last = (l == _NL - 1)
        op_ks = [1, 2, 4]
        dev_ks = [0, 3]
        opf_n = _combine(opf, [gs[k] for k in op_ks], [es[k] for k in op_ks],
                         [gconv_W[l, k, :_H] for k in op_ks],
                         [gconv_W[l, k, _H:] for k in op_ks],
                         [gconv_b[l, k] for k in op_ks], last, bm=2000)
        devf_n = _combine(devf, [gs[k] for k in dev_ks], [es[k] for k in dev_ks],
                          [gconv_W[l, k, :_H] for k in dev_ks],
                          [gconv_W[l, k, _H:] for k in dev_ks],
                          [gconv_b[l, k] for k in dev_ks], last, bm=1024)
        opf, devf = opf_n, devf_n

    wa = jnp.zeros((64, 16), jnp.float32)
    wa = wa.at[:, :3].set(fp_W[:_H])
    wa = wa.at[:, 3:4].set(fn_W)
    ba = jnp.zeros((16,), jnp.float32).at[3].set(fn_b[0])
    a_tab = _proj_head(opf, wa, ba, bm=2000)
    wb = jnp.zeros((64, 16), jnp.float32)
    wb = wb.at[:, :3].set(fp_W[_H + _EH:])
    b_tab = _proj_head(devf, wb, jnp.zeros((16,), jnp.float32), bm=1024)

    ga = pad_rows(_gather_rows(a_tab, place_edge_index[0]))
    gb = pad_rows(_gather_rows(b_tab, place_edge_index[1]))

    wfin = jnp.zeros((16, 16), jnp.float32)
    wfin = wfin.at[:_EH, :3].set(fp_W[_H:_H + _EH])
    wfin = wfin.at[_EH, :3].set(fp_b)
    d16 = _final_edge(ga, gb, ef[3], wfin, bm=4096)
    d = d16[:e_p, :3]
    nccl = a_tab[:, 3]
    return (d, nccl)


# speedup vs baseline: 3.0141x; 2.1829x over previous
import functools
import jax
import jax.numpy as jnp
from jax import lax
from jax.experimental import pallas as pl
from jax.experimental.pallas import tpu as pltpu
from jax.experimental.pallas import tpu_sc as plsc

_NOP = 50000
_NDEV = 1024
_H = 64
_EH = 8
_NL = 6

_NC = 2
_NT = 16
_NOPP = _NOP + 48
_NDEVP = _NDEV + 128
_EPM = 2 * _NT * 8 * 128


def _elu(x):
    return jnp.where(x > 0, x, jnp.exp(jnp.minimum(x, 0.0)) - 1.0)


def _init_nodes_body(x_ref, w_ref, b_ref, o_ref):
    y = jnp.dot(x_ref[...], w_ref[0], preferred_element_type=jnp.float32)
    o_ref[0] = _elu(y + b_ref[0])


def _init_nodes(x, w, b, bm):
    n, din = x.shape
    nb = n // bm
    wst = w.reshape(din, 2, 32).transpose(1, 0, 2)
    bst = b.reshape(2, 1, 32)
    return pl.pallas_call(
        _init_nodes_body,
        grid=(2, nb),
        in_specs=[
            pl.BlockSpec((bm, din), lambda h, i: (i, 0)),
            pl.BlockSpec((1, din, 32), lambda h, i: (h, 0, 0)),
            pl.BlockSpec((1, 1, 32), lambda h, i: (h, 0, 0)),
        ],
        out_specs=pl.BlockSpec((1, bm, 32), lambda h, i: (h, i, 0)),
        out_shape=jax.ShapeDtypeStruct((2, n, 32), jnp.float32),
    )(x, wst, bst)


def _edge_mlp_body(x_ref, w_ref, b_ref, o_ref):
    z = jnp.dot(x_ref[...], w_ref[...], preferred_element_type=jnp.float32)
    z = z + b_ref[0]
    col = lax.broadcasted_iota(jnp.int32, z.shape, 1)
    o_ref[...] = jnp.where(col < _EH, _elu(z),
                           jnp.where(col == _EH, 1.0, 0.0))


def _edge_mlp(raw_pad, w, b, bm):
    ep = raw_pad.shape[0]
    w16 = jnp.pad(w, ((0, 0), (0, 16 - _EH)))
    b16 = jnp.pad(b, (0, 16 - _EH)).reshape(1, 16)
    return pl.pallas_call(
        _edge_mlp_body,
        grid=(ep // bm,),
        in_specs=[
            pl.BlockSpec((bm, 16), lambda i: (i, 0)),
            pl.BlockSpec((16, 16), lambda i: (0, 0)),
            pl.BlockSpec((1, 16), lambda i: (0, 0)),
        ],
        out_specs=pl.BlockSpec((bm, 16), lambda i: (i, 0)),
        out_shape=jax.ShapeDtypeStruct((ep, 16), jnp.float32),
    )(raw_pad, w16, b16)


def _combine_body(nark, op_ref,
                  g0l, g0h, g1l, g1h, g2l, g2h,
                  e0a, e0b, e1a, e1b, e2a, e2b,
                  w0, w1, w2, we0, we1, we2, o_ref,
                  *, last):
    gl = (g0l, g1l, g2l)
    gh = (g0h, g1h, g2h)
    ea = (e0a, e1a, e2a)
    eb = (e0b, e1b, e2b)
    ws = (w0, w1, w2)
    wes = (we0, we1, we2)
    acc = jnp.zeros(op_ref.shape[1:], jnp.float32)
    for k in range(nark):
        es = ea[k][0] + eb[k][0]
        cnt = es[:, _EH:_EH + 1]
        num = jnp.dot(gl[k][0], ws[k][0, 0], preferred_element_type=jnp.float32)
        num += jnp.dot(gh[k][0], ws[k][0, 1], preferred_element_type=jnp.float32)
        num += jnp.dot(es, wes[k][0], preferred_element_type=jnp.float32)
        acc += num / jnp.maximum(cnt, 1.0)
    o = op_ref[0] + acc * (1.0 / nark)
    o_ref[0] = o if last else _elu(o)


def _combine(opf, gs, ess, whs, wes, bs, last, bm):
    nark = len(gs)
    n = opf.shape[1]
    nb = n // bm
    wsts, wests = [], []
    for k in range(nark):
        w = whs[k].reshape(2, 32, 2, 32).transpose(2, 0, 1, 3)
        wsts.append(w)
        wep = jnp.zeros((16, 64), jnp.float32)
        wep = wep.at[:_EH].set(wes[k])
        wep = wep.at[_EH].set(bs[k])
        wests.append(wep.reshape(16, 2, 32).transpose(1, 0, 2))
    big = pl.BlockSpec((1, bm, 32), lambda h, i: (0, i, 0))
    bigh = pl.BlockSpec((1, bm, 32), lambda h, i: (1, i, 0))
    esa = pl.BlockSpec((1, bm, 16), lambda h, i: (0, i, 0))
    esb = pl.BlockSpec((1, bm, 16), lambda h, i: (1, i, 0))
    wsp = pl.BlockSpec((1, 2, 32, 32), lambda h, i: (h, 0, 0, 0))
    wesp = pl.BlockSpec((1, 16, 32), lambda h, i: (h, 0, 0))
    g3 = [gs[k] if k < nark else gs[0] for k in range(3)]
    e3 = [ess[k] if k < nark else ess[0] for k in range(3)]
    w3 = [wsts[k] if k < nark else wsts[0] for k in range(3)]
    we3 = [wests[k] if k < nark else wests[0] for k in range(3)]
    in_specs = [pl.BlockSpec((1, bm, 32), lambda h, i: (h, i, 0))]
    args = [opf]
    for k in range(3):
        in_specs += [big, bigh]
        args += [g3[k], g3[k]]
    for k in range(3):
        in_specs += [esa, esb]
        args += [e3[k], e3[k]]
    in_specs += [wsp] * 3 + [wesp] * 3
    args += w3 + we3
    return pl.pallas_call(
        functools.partial(_combine_body, len(gs), last=last),
        grid=(2, nb),
        in_specs=in_specs,
        out_specs=pl.BlockSpec((1, bm, 32), lambda h, i: (h, i, 0)),
        out_shape=jax.ShapeDtypeStruct((2, n, 32), jnp.float32),
    )(*args)


def _proj_head_body(xl_ref, xh_ref, w_ref, b_ref, o_ref):
    y = jnp.dot(xl_ref[0], w_ref[:32], preferred_element_type=jnp.float32)
    y += jnp.dot(xh_ref[0], w_ref[32:], preferred_element_type=jnp.float32)
    o_ref[...] = y + b_ref[...]


def _proj_head(opf, w64x16, b16, bm):
    n = opf.shape[1]
    return pl.pallas_call(
        _proj_head_body,
        grid=(n // bm,),
        in_specs=[
            pl.BlockSpec((1, bm, 32), lambda i: (0, i, 0)),
            pl.BlockSpec((1, bm, 32), lambda i: (1, i, 0)),
            pl.BlockSpec((64, 16), lambda i: (0, 0)),
            pl.BlockSpec((1, 16), lambda i: (0, 0)),
        ],
        out_specs=pl.BlockSpec((bm, 16), lambda i: (i, 0)),
        out_shape=jax.ShapeDtypeStruct((n, 16), jnp.float32),
    )(opf, opf, w64x16, b16.reshape(1, 16))


def _final_edge_body(ga_ref, gb_ref, ef_ref, w_ref, o_ref):
    y = ga_ref[...] + gb_ref[...]
    y += jnp.dot(ef_ref[...], w_ref[...], preferred_element_type=jnp.float32)
    o_ref[...] = y


def _final_edge(ga, gb, ef3, wfin, bm):
    ep = ga.shape[0]
    return pl.pallas_call(
        _final_edge_body,
        grid=(ep // bm,),
        in_specs=[
            pl.BlockSpec((bm, 16), lambda i: (i, 0)),
            pl.BlockSpec((bm, 16), lambda i: (i, 0)),
            pl.BlockSpec((bm, 16), lambda i: (i, 0)),
            pl.BlockSpec((16, 16), lambda i: (0, 0)),
        ],
        out_specs=pl.BlockSpec((bm, 16), lambda i: (i, 0)),
        out_shape=jax.ShapeDtypeStruct((ep, 16), jnp.float32),
    )(ga, gb, ef3, wfin)


_SC_MESH = dict(core_axis_name="c", subcore_axis_name="s",
                num_cores=_NC, num_subcores=_NT)


def _sc_g_kernel(opf2, devf2, z32, idxs):
    rt = idxs[1][1].shape[0]
    rp = idxs[3][1].shape[0]
    rl = idxs[0][1].shape[0]
    op_sds = jax.ShapeDtypeStruct((2 * _NOPP, 32), jnp.float32)
    dev_sds = jax.ShapeDtypeStruct((2 * _NDEVP, 32), jnp.float32)

    @functools.partial(
        pl.kernel,
        mesh=plsc.VectorSubcoreMesh(**_SC_MESH),
        compiler_params=pltpu.CompilerParams(use_tc_tiling_on_sc=False),
        out_type=(op_sds, op_sds, op_sds, dev_sds, dev_sds),
        scratch_types=[
            pltpu.VMEM_SHARED((_NOPP, 32), jnp.float32),
            pltpu.VMEM_SHARED((_NDEVP, 32), jnp.float32),
            pltpu.VMEM((8, 128), jnp.int32),
            pltpu.VMEM((8, 128), jnp.int32),
            pltpu.VMEM((4, 128, 32), jnp.float32),
            pltpu.SemaphoreType.DMA,
        ],
    )
    def k(opf2_h, devf2_h, z32_h,
          s1_h, d1_h, s2_h, d2_h, s4_h, d4_h, s0_h, d0_h, s3_h, d3_h,
          g1_h, g2_h, g4_h, g0_h, g3_h,
          acc_op, acc_dev, sidx, didx, gbuf, sem):
        c = lax.axis_index("c")
        s = lax.axis_index("s")

        def scatter_etype(tab_h, acc, s_h, d_h, rows):
            nb = rows // (_NT * 8)

            def body(bi, carry):
                rbase = (s * nb + bi) * 8
                pltpu.sync_copy(s_h.at[pl.ds(c * rows + rbase, 8)], sidx)
                pltpu.sync_copy(d_h.at[pl.ds(rbase, 8)], didx)
                for r in range(2):
                    hs = [pltpu.async_copy(tab_h.at[sidx.at[4 * r + j]],
                                           gbuf.at[j], sem)
                          for j in range(4)]
                    for j in range(4):
                        hs[j].wait()
                    for j in range(4):
                        pltpu.sync_copy(gbuf.at[j],
                                        acc.at[didx.at[4 * r + j]], add=True)
                return carry

            lax.fori_loop(0, nb, body, 0)

        nop_pt = _NOPP // _NT
        ndev_pt = _NDEVP // _NT

        def zero(acc, npt):
            pltpu.sync_copy(z32_h.at[pl.ds(0, npt)],
                            acc.at[pl.ds(s * npt, npt)])

        def evac(acc, g_h, npt, accrows):
            pltpu.sync_copy(acc.at[pl.ds(s * npt, npt)],
                            g_h.at[pl.ds(c * accrows + s * npt, npt)])

        zero(acc_op, nop_pt)
        zero(acc_dev, ndev_pt)
        plsc.subcore_barrier()
        scatter_etype(opf2_h, acc_op, s1_h, d1_h, rt)
        plsc.subcore_barrier()
        evac(acc_op, g1_h, nop_pt, _NOPP)
        zero(acc_op, nop_pt)
        plsc.subcore_barrier()
        scatter_etype(opf2_h, acc_op, s2_h, d2_h, rt)
        plsc.subcore_barrier()
        evac(acc_op, g2_h, nop_pt, _NOPP)
        zero(acc_op, nop_pt)
        plsc.subcore_barrier()
        scatter_etype(devf2_h, acc_op, s4_h, d4_h, rp)
        scatter_etype(devf2_h, acc_dev, s0_h, d0_h, rl)
        plsc.subcore_barrier()
        evac(acc_op, g4_h, nop_pt, _NOPP)
        evac(acc_dev, g0_h, ndev_pt, _NDEVP)
        zero(acc_dev, ndev_pt)
        plsc.subcore_barrier()
        scatter_etype(opf2_h, acc_dev, s3_h, d3_h, rp)
        plsc.subcore_barrier()
        evac(acc_dev, g3_h, ndev_pt, _NDEVP)

    g1, g2, g4, g0, g3 = k(
        opf2, devf2, z32,
        idxs[1][0], idxs[1][1], idxs[2][0], idxs[2][1],
        idxs[4][0], idxs[4][1], idxs[0][0], idxs[0][1],
        idxs[3][0], idxs[3][1])
    return [g0.reshape(2, _NDEVP, 32), g1.reshape(2, _NOPP, 32),
            g2.reshape(2, _NOPP, 32), g3.reshape(2, _NDEVP, 32),
            g4.reshape(2, _NOPP, 32)]


def _sc_es_kernel(efs, didxs, z16):
    op_sds = jax.ShapeDtypeStruct((2 * _NOPP, 16), jnp.float32)
    dev_sds = jax.ShapeDtypeStruct((2 * _NDEVP, 16), jnp.float32)

    @functools.partial(
        pl.kernel,
        mesh=plsc.VectorSubcoreMesh(**_SC_MESH),
        compiler_params=pltpu.CompilerParams(use_tc_tiling_on_sc=False),
        out_type=(dev_sds, op_sds, op_sds, dev_sds, op_sds),
        scratch_types=[
            pltpu.VMEM_SHARED((_NOPP, 16), jnp.float32),
            pltpu.VMEM_SHARED((_NDEVP, 16), jnp.float32),
            pltpu.VMEM((8, 128), jnp.int32),
            pltpu.VMEM((8, 128, 16), jnp.float32),
            pltpu.SemaphoreType.DMA,
        ],
    )
    def k(ef0_h, ef1_h, ef2_h, ef3_h, ef4_h,
          d0_h, d1_h, d2_h, d3_h, d4_h, z16_h,
          e0_h, e1_h, e2_h, e3_h, e4_h,
          acc_op, acc_dev, didx, ebuf, sem):
        c = lax.axis_index("c")
        s = lax.axis_index("s")
        w = c * _NT + s

        def scatter_etype(ef_h, acc, d_h, rows):
            nb = rows // (2 * _NT * 8)

            def body(bi, carry):
                rbase = (w * nb + bi) * 8
                pltpu.sync_copy(d_h.at[pl.ds(rbase, 8)], didx)
                hs = [pltpu.async_copy(
                    ef_h.at[pl.ds((rbase + j) * 128, 128)], ebuf.at[j], sem)
                    for j in range(8)]
                for j in range(8):
                    hs[j].wait()
                for j in range(8):
                    pltpu.sync_copy(ebuf.at[j], acc.at[didx.at[j]], add=True)
                return carry

            lax.fori_loop(0, nb, body, 0)

        nop_pt = _NOPP // _NT
        ndev_pt = _NDEVP // _NT

        def zero(acc, npt):
            pltpu.sync_copy(z16_h.at[pl.ds(0, npt)],
                            acc.at[pl.ds(s * npt, npt)])

        def evac(acc, e_h, npt, accrows):
            pltpu.sync_copy(acc.at[pl.ds(s * npt, npt)],
                            e_h.at[pl.ds(c * accrows + s * npt, npt)])

        zero(acc_op, nop_pt)
        zero(acc_dev, ndev_pt)
        plsc.subcore_barrier()
        scatter_etype(ef1_h, acc_op, d1_h, d1_h.shape[0])
        plsc.subcore_barrier()
        evac(acc_op, e1_h, nop_pt, _NOPP)
        zero(acc_op, nop_pt)
        plsc.subcore_barrier()
        scatter_etype(ef2_h, acc_op, d2_h, d2_h.shape[0])
        plsc.subcore_barrier()
        evac(acc_op, e2_h, nop_pt, _NOPP)
        zero(acc_op, nop_pt)
        plsc.subcore_barrier()
        scatter_etype(ef4_h, acc_op, d4_h, d4_h.shape[0])
        scatter_etype(ef0_h, acc_dev, d0_h, d0_h.shape[0])
        plsc.subcore_barrier()
        evac(acc_op, e4_h, nop_pt, _NOPP)
        evac(acc_dev, e0_h, ndev_pt, _NDEVP)
        zero(acc_dev, ndev_pt)
        plsc.subcore_barrier()
        scatter_etype(ef3_h, acc_dev, d3_h, d3_h.shape[0])
        plsc.subcore_barrier()
        evac(acc_dev, e3_h, ndev_pt, _NDEVP)

    es0, es1, es2, es3, es4 = k(efs[0], efs[1], efs[2], efs[3], efs[4],
                                didxs[0], didxs[1], didxs[2], didxs[3],
                                didxs[4], z16)
    return [es0.reshape(2, _NDEVP, 16), es1.reshape(2, _NOPP, 16),
            es2.reshape(2, _NOPP, 16), es3.reshape(2, _NDEVP, 16),
            es4.reshape(2, _NOPP, 16)]


def _sc_final_gather(a_tab, b_tab, ps_rows, pd_rows):
    rp = ps_rows.shape[0]
    out_sds = jax.ShapeDtypeStruct((rp * 128, 16), jnp.float32)

    @functools.partial(
        pl.kernel,
        mesh=plsc.VectorSubcoreMesh(**_SC_MESH),
        compiler_params=pltpu.CompilerParams(use_tc_tiling_on_sc=False),
        out_type=(out_sds, out_sds),
        scratch_types=[
            pltpu.VMEM((8, 128), jnp.int32),
            pltpu.VMEM((8, 128, 16), jnp.float32),
            pltpu.SemaphoreType.DMA,
        ],
    )
    def k(a_h, b_h, ps_h, pd_h, ga_h, gb_h, idx, buf, sem):
        c = lax.axis_index("c")
        s = lax.axis_index("s")
        w = c * _NT + s
        nb = rp // (2 * _NT * 8)

        def gather_tab(tab_h, i_h, o_h):
            def body(bi, carry):
                rbase = (w * nb + bi) * 8
                pltpu.sync_copy(i_h.at[pl.ds(rbase, 8)], idx)
                hs = [pltpu.async_copy(tab_h.at[idx.at[j]], buf.at[j], sem)
                      for j in range(8)]
                for j in range(8):
                    hs[j].wait()
                for j in range(8):
                    pltpu.sync_copy(buf.at[j],
                                    o_h.at[pl.ds((rbase + j) * 128, 128)])
                return carry

            lax.fori_loop(0, nb, body, 0)

        gather_tab(a_h, ps_h, ga_h)
        gather_tab(b_h, pd_h, gb_h)

    return k(a_tab, b_tab, ps_rows, pd_rows)


def kernel(op_feats, device_feats, tensor_feats, link_feats, place_feats,
           prev_edge_index, link_edge_index, place_edge_index,
           op_W, op_b, dev_W, dev_b, et_W, et_b, gconv_W, gconv_b,
           fp_W, fp_b, fn_W, fn_b):
    e_p = place_feats.shape[0]

    def pad_rows(x, m=_EPM):
        r = (-x.shape[0]) % m
        return x if r == 0 else jnp.pad(x, ((0, r), (0, 0)))

    def pad_idx(idx, pad_val):
        e = idx.shape[0]
        ep = -(-e // _EPM) * _EPM
        a = jnp.full((ep,), pad_val, jnp.int32).at[:e].set(idx)
        return a.reshape(ep // 128, 128)

    opf = _init_nodes(op_feats, op_W, op_b, bm=2000)
    devf = _init_nodes(device_feats, dev_W, dev_b, bm=1024)

    ef = [None] * 5
    ef[0] = _edge_mlp(pad_rows(link_feats), et_W[0], et_b[0], bm=4096)
    ef[1] = _edge_mlp(pad_rows(tensor_feats), et_W[1], et_b[1], bm=4096)
    ef[2] = _edge_mlp(pad_rows(tensor_feats), et_W[2], et_b[2], bm=4096)
    ef[3] = _edge_mlp(pad_rows(place_feats), et_W[3], et_b[3], bm=4096)
    ef[4] = _edge_mlp(pad_rows(place_feats), et_W[4], et_b[4], bm=4096)

    src_idx = [link_edge_index[0], prev_edge_index[0], prev_edge_index[1],
               place_edge_index[0], place_edge_index[1]]
    dst_idx = [link_edge_index[1], prev_edge_index[1], prev_edge_index[0],
               place_edge_index[1], place_edge_index[0]]
    src_is_op = [False, True, True, True, False]
    dst_is_op = [False, True, True, False, True]

    idxs = []
    didxs = []
    for i in range(5):
        s_off = _NOP if src_is_op[i] else _NDEV
        d_dum = _NOP if dst_is_op[i] else _NDEV
        srows = pad_idx(src_idx[i], 0)
        gidx = jnp.concatenate([srows, srows + s_off], axis=0)
        didx = pad_idx(dst_idx[i], d_dum)
        idxs.append((gidx, didx))
        didxs.append(didx)

    z32 = jnp.zeros((_NOPP // _NT, 32), jnp.float32)
    z16 = jnp.zeros((_NOPP // _NT, 16), jnp.float32)

    es = _sc_es_kernel(ef, didxs, z16)

    for l in range(_NL):
        gs = _sc_g_kernel(opf.reshape(2 * _NOP, 32),
                          devf.reshape(2 * _NDEV, 32), z32, idxs)
        last = (l == _NL - 1)
        op_ks = [1, 2, 4]
        dev_ks = [0, 3]
        opf_n = _combine(opf, [gs[k] for k in op_ks], [es[k] for k in op_ks],
                         [gconv_W[l, k, :_H] for k in op_ks],
                         [gconv_W[l, k, _H:] for k in op_ks],
                         [gconv_b[l, k] for k in op_ks], last, bm=2000)
        devf_n = _combine(devf, [gs[k] for k in dev_ks], [es[k] for k in dev_ks],
                          [gconv_W[l, k, :_H] for k in dev_ks],
                          [gconv_W[l, k, _H:] for k in dev_ks],
                          [gconv_b[l, k] for k in dev_ks], last, bm=1024)
        opf, devf = opf_n, devf_n

    wa = jnp.zeros((64, 16), jnp.float32)
    wa = wa.at[:, :3].set(fp_W[:_H])
    wa = wa.at[:, 3:4].set(fn_W)
    ba = jnp.zeros((16,), jnp.float32).at[3].set(fn_b[0])
    a_tab = _proj_head(opf, wa, ba, bm=2000)
    wb = jnp.zeros((64, 16), jnp.float32)
    wb = wb.at[:, :3].set(fp_W[_H + _EH:])
    b_tab = _proj_head(devf, wb, jnp.zeros((16,), jnp.float32), bm=1024)

    ga, gb = _sc_final_gather(a_tab, b_tab,
                              pad_idx(place_edge_index[0], 0),
                              pad_idx(place_edge_index[1], 0))

    wfin = jnp.zeros((16, 16), jnp.float32)
    wfin = wfin.at[:_EH, :3].set(fp_W[_H:_H + _EH])
    wfin = wfin.at[_EH, :3].set(fp_b)
    d16 = _final_edge(ga, gb, ef[3], wfin, bm=4096)
    d = d16[:e_p, :3]
    nccl = a_tab[:, 3]
    return (d, nccl)


# speedup vs baseline: 3.1078x; 1.0311x over previous
import functools
import jax
import jax.numpy as jnp
from jax import lax
from jax.experimental import pallas as pl
from jax.experimental.pallas import tpu as pltpu
from jax.experimental.pallas import tpu_sc as plsc

_NOP = 50000
_NDEV = 1024
_H = 64
_EH = 8
_NL = 6

_NC = 2
_NT = 16
_NOPP = _NOP + 48
_NDEVP = _NDEV + 128
_EPM = 2 * _NT * 8 * 128


def _elu(x):
    return jnp.where(x > 0, x, jnp.exp(jnp.minimum(x, 0.0)) - 1.0)


def _init_nodes_body(x_ref, w_ref, b_ref, o_ref):
    y = jnp.dot(x_ref[...], w_ref[0], preferred_element_type=jnp.float32)
    o_ref[0] = _elu(y + b_ref[0])


def _init_nodes(x, w, b, bm):
    n, din = x.shape
    nb = n // bm
    wst = w.reshape(din, 2, 32).transpose(1, 0, 2)
    bst = b.reshape(2, 1, 32)
    return pl.pallas_call(
        _init_nodes_body,
        grid=(2, nb),
        in_specs=[
            pl.BlockSpec((bm, din), lambda h, i: (i, 0)),
            pl.BlockSpec((1, din, 32), lambda h, i: (h, 0, 0)),
            pl.BlockSpec((1, 1, 32), lambda h, i: (h, 0, 0)),
        ],
        out_specs=pl.BlockSpec((1, bm, 32), lambda h, i: (h, i, 0)),
        out_shape=jax.ShapeDtypeStruct((2, n, 32), jnp.float32),
    )(x, wst, bst)


def _edge_mlp_body(x_ref, w_ref, b_ref, o_ref):
    z = jnp.dot(x_ref[...], w_ref[...], preferred_element_type=jnp.float32)
    z = z + b_ref[0]
    col = lax.broadcasted_iota(jnp.int32, z.shape, 1)
    o_ref[...] = jnp.where(col < _EH, _elu(z),
                           jnp.where(col == _EH, 1.0, 0.0))


def _edge_mlp(raw_pad, w, b, bm):
    ep = raw_pad.shape[0]
    w16 = jnp.pad(w, ((0, 0), (0, 16 - _EH)))
    b16 = jnp.pad(b, (0, 16 - _EH)).reshape(1, 16)
    return pl.pallas_call(
        _edge_mlp_body,
        grid=(ep // bm,),
        in_specs=[
            pl.BlockSpec((bm, 16), lambda i: (i, 0)),
            pl.BlockSpec((16, 16), lambda i: (0, 0)),
            pl.BlockSpec((1, 16), lambda i: (0, 0)),
        ],
        out_specs=pl.BlockSpec((bm, 16), lambda i: (i, 0)),
        out_shape=jax.ShapeDtypeStruct((ep, 16), jnp.float32),
    )(raw_pad, w16, b16)


def _combine_body(nark, op_ref,
                  g0l, g0h, g1l, g1h, g2l, g2h,
                  e0a, e0b, e1a, e1b, e2a, e2b,
                  w0, w1, w2, we0, we1, we2, o_ref,
                  *, last):
    gl = (g0l, g1l, g2l)
    gh = (g0h, g1h, g2h)
    ea = (e0a, e1a, e2a)
    eb = (e0b, e1b, e2b)
    ws = (w0, w1, w2)
    wes = (we0, we1, we2)
    acc = jnp.zeros(op_ref.shape[1:], jnp.float32)
    for k in range(nark):
        es = ea[k][0] + eb[k][0]
        cnt = es[:, _EH:_EH + 1]
        num = jnp.dot(gl[k][0], ws[k][0, 0], preferred_element_type=jnp.float32)
        num += jnp.dot(gh[k][0], ws[k][0, 1], preferred_element_type=jnp.float32)
        num += jnp.dot(es, wes[k][0], preferred_element_type=jnp.float32)
        acc += num / jnp.maximum(cnt, 1.0)
    o = op_ref[0] + acc * (1.0 / nark)
    o_ref[0] = o if last else _elu(o)


def _combine(opf, gs, ess, whs, wes, bs, last, bm):
    nark = len(gs)
    n = opf.shape[1]
    nb = n // bm
    wsts, wests = [], []
    for k in range(nark):
        w = whs[k].reshape(2, 32, 2, 32).transpose(2, 0, 1, 3)
        wsts.append(w)
        wep = jnp.zeros((16, 64), jnp.float32)
        wep = wep.at[:_EH].set(wes[k])
        wep = wep.at[_EH].set(bs[k])
        wests.append(wep.reshape(16, 2, 32).transpose(1, 0, 2))
    big = pl.BlockSpec((1, bm, 32), lambda h, i: (0, i, 0))
    bigh = pl.BlockSpec((1, bm, 32), lambda h, i: (1, i, 0))
    esa = pl.BlockSpec((1, bm, 16), lambda h, i: (0, i, 0))
    esb = pl.BlockSpec((1, bm, 16), lambda h, i: (1, i, 0))
    wsp = pl.BlockSpec((1, 2, 32, 32), lambda h, i: (h, 0, 0, 0))
    wesp = pl.BlockSpec((1, 16, 32), lambda h, i: (h, 0, 0))
    g3 = [gs[k] if k < nark else gs[0] for k in range(3)]
    e3 = [ess[k] if k < nark else ess[0] for k in range(3)]
    w3 = [wsts[k] if k < nark else wsts[0] for k in range(3)]
    we3 = [wests[k] if k < nark else wests[0] for k in range(3)]
    in_specs = [pl.BlockSpec((1, bm, 32), lambda h, i: (h, i, 0))]
    args = [opf]
    for k in range(3):
        in_specs += [big, bigh]
        args += [g3[k], g3[k]]
    for k in range(3):
        in_specs += [esa, esb]
        args += [e3[k], e3[k]]
    in_specs += [wsp] * 3 + [wesp] * 3
    args += w3 + we3
    return pl.pallas_call(
        functools.partial(_combine_body, len(gs), last=last),
        grid=(2, nb),
        in_specs=in_specs,
        out_specs=pl.BlockSpec((1, bm, 32), lambda h, i: (h, i, 0)),
        out_shape=jax.ShapeDtypeStruct((2, n, 32), jnp.float32),
    )(*args)


def _proj_head_body(xl_ref, xh_ref, w_ref, b_ref, o_ref):
    y = jnp.dot(xl_ref[0], w_ref[:32], preferred_element_type=jnp.float32)
    y += jnp.dot(xh_ref[0], w_ref[32:], preferred_element_type=jnp.float32)
    o_ref[...] = y + b_ref[...]


def _proj_head(opf, w64x16, b16, bm):
    n = opf.shape[1]
    return pl.pallas_call(
        _proj_head_body,
        grid=(n // bm,),
        in_specs=[
            pl.BlockSpec((1, bm, 32), lambda i: (0, i, 0)),
            pl.BlockSpec((1, bm, 32), lambda i: (1, i, 0)),
            pl.BlockSpec((64, 16), lambda i: (0, 0)),
            pl.BlockSpec((1, 16), lambda i: (0, 0)),
        ],
        out_specs=pl.BlockSpec((bm, 16), lambda i: (i, 0)),
        out_shape=jax.ShapeDtypeStruct((n, 16), jnp.float32),
    )(opf, opf, w64x16, b16.reshape(1, 16))


def _final_edge_body(ga_ref, gb_ref, ef_ref, w_ref, o_ref):
    y = ga_ref[...] + gb_ref[...]
    y += jnp.dot(ef_ref[...], w_ref[...], preferred_element_type=jnp.float32)
    o_ref[...] = y


def _final_edge(ga, gb, ef3, wfin, bm):
    ep = ga.shape[0]
    return pl.pallas_call(
        _final_edge_body,
        grid=(ep // bm,),
        in_specs=[
            pl.BlockSpec((bm, 16), lambda i: (i, 0)),
            pl.BlockSpec((bm, 16), lambda i: (i, 0)),
            pl.BlockSpec((bm, 16), lambda i: (i, 0)),
            pl.BlockSpec((16, 16), lambda i: (0, 0)),
        ],
        out_specs=pl.BlockSpec((bm, 16), lambda i: (i, 0)),
        out_shape=jax.ShapeDtypeStruct((ep, 16), jnp.float32),
    )(ga, gb, ef3, wfin)


_SC_MESH = dict(core_axis_name="c", subcore_axis_name="s",
                num_cores=_NC, num_subcores=_NT)


def _sc_g_kernel(opf2, devf2, z32, idxs):
    rt = idxs[1][1].shape[0]
    rp = idxs[3][1].shape[0]
    rl = idxs[0][1].shape[0]
    op_sds = jax.ShapeDtypeStruct((2 * _NOPP, 32), jnp.float32)
    dev_sds = jax.ShapeDtypeStruct((2 * _NDEVP, 32), jnp.float32)

    @functools.partial(
        pl.kernel,
        mesh=plsc.VectorSubcoreMesh(**_SC_MESH),
        compiler_params=pltpu.CompilerParams(use_tc_tiling_on_sc=False),
        out_type=(op_sds, op_sds, op_sds, dev_sds, dev_sds),
        scratch_types=[
            pltpu.VMEM_SHARED((_NOPP, 32), jnp.float32),
            pltpu.VMEM_SHARED((_NDEVP, 32), jnp.float32),
            pltpu.VMEM((8, 128), jnp.int32),
            pltpu.VMEM((8, 128), jnp.int32),
            pltpu.VMEM((4, 128, 32), jnp.float32),
            pltpu.SemaphoreType.DMA,
            pltpu.SemaphoreType.DMA,
            pltpu.SemaphoreType.DMA,
            pltpu.SemaphoreType.DMA,
        ],
    )
    def k(opf2_h, devf2_h, z32_h,
          s1_h, d1_h, s2_h, d2_h, s4_h, d4_h, s0_h, d0_h, s3_h, d3_h,
          g1_h, g2_h, g4_h, g0_h, g3_h,
          acc_op, acc_dev, sidx, didx, gbuf, sm0, sm1, sm2, sm3):
        sems = (sm0, sm1, sm2, sm3)
        c = lax.axis_index("c")
        s = lax.axis_index("s")

        def scatter_etype(tab_h, acc, s_h, d_h, rows):
            nb = rows // (_NT * 8)

            def body(bi, carry):
                rbase = (s * nb + bi) * 8
                pltpu.sync_copy(s_h.at[pl.ds(c * rows + rbase, 8)], sidx)
                pltpu.sync_copy(d_h.at[pl.ds(rbase, 8)], didx)

                def fire_g(j):
                    return pltpu.async_copy(tab_h.at[sidx.at[j]],
                                            gbuf.at[j % 4], sems[j % 4])

                def fire_s(j):
                    return pltpu.async_copy(gbuf.at[j % 4],
                                            acc.at[didx.at[j]],
                                            sems[j % 4], add=True)

                hg = {j: fire_g(j) for j in range(4)}
                hs = {}
                for j in range(8):
                    hg[j].wait()
                    hs[j] = fire_s(j)
                    if 1 <= j <= 4:
                        hs[j - 1].wait()
                        hg[j + 3] = fire_g(j + 3)
                for j in range(4, 8):
                    hs[j].wait()
                return carry

            lax.fori_loop(0, nb, body, 0)

        nop_pt = _NOPP // _NT
        ndev_pt = _NDEVP // _NT

        def zero(acc, npt):
            pltpu.sync_copy(z32_h.at[pl.ds(0, npt)],
                            acc.at[pl.ds(s * npt, npt)])

        def evac(acc, g_h, npt, accrows):
            pltpu.sync_copy(acc.at[pl.ds(s * npt, npt)],
                            g_h.at[pl.ds(c * accrows + s * npt, npt)])

        zero(acc_op, nop_pt)
        zero(acc_dev, ndev_pt)
        plsc.subcore_barrier()
        scatter_etype(opf2_h, acc_op, s1_h, d1_h, rt)
        plsc.subcore_barrier()
        evac(acc_op, g1_h, nop_pt, _NOPP)
        zero(acc_op, nop_pt)
        plsc.subcore_barrier()
        scatter_etype(opf2_h, acc_op, s2_h, d2_h, rt)
        plsc.subcore_barrier()
        evac(acc_op, g2_h, nop_pt, _NOPP)
        zero(acc_op, nop_pt)
        plsc.subcore_barrier()
        scatter_etype(devf2_h, acc_op, s4_h, d4_h, rp)
        scatter_etype(devf2_h, acc_dev, s0_h, d0_h, rl)
        plsc.subcore_barrier()
        evac(acc_op, g4_h, nop_pt, _NOPP)
        evac(acc_dev, g0_h, ndev_pt, _NDEVP)
        zero(acc_dev, ndev_pt)
        plsc.subcore_barrier()
        scatter_etype(opf2_h, acc_dev, s3_h, d3_h, rp)
        plsc.subcore_barrier()
        evac(acc_dev, g3_h, ndev_pt, _NDEVP)

    g1, g2, g4, g0, g3 = k(
        opf2, devf2, z32,
        idxs[1][0], idxs[1][1], idxs[2][0], idxs[2][1],
        idxs[4][0], idxs[4][1], idxs[0][0], idxs[0][1],
        idxs[3][0], idxs[3][1])
    return [g0.reshape(2, _NDEVP, 32), g1.reshape(2, _NOPP, 32),
            g2.reshape(2, _NOPP, 32), g3.reshape(2, _NDEVP, 32),
            g4.reshape(2, _NOPP, 32)]


def _sc_es_kernel(efs, didxs, z16):
    op_sds = jax.ShapeDtypeStruct((2 * _NOPP, 16), jnp.float32)
    dev_sds = jax.ShapeDtypeStruct((2 * _NDEVP, 16), jnp.float32)

    @functools.partial(
        pl.kernel,
        mesh=plsc.VectorSubcoreMesh(**_SC_MESH),
        compiler_params=pltpu.CompilerParams(use_tc_tiling_on_sc=False),
        out_type=(dev_sds, op_sds, op_sds, dev_sds, op_sds),
        scratch_types=[
            pltpu.VMEM_SHARED((_NOPP, 16), jnp.float32),
            pltpu.VMEM_SHARED((_NDEVP, 16), jnp.float32),
            pltpu.VMEM((8, 128), jnp.int32),
            pltpu.VMEM((8, 128, 16), jnp.float32),
            pltpu.SemaphoreType.DMA,
        ],
    )
    def k(ef0_h, ef1_h, ef2_h, ef3_h, ef4_h,
          d0_h, d1_h, d2_h, d3_h, d4_h, z16_h,
          e0_h, e1_h, e2_h, e3_h, e4_h,
          acc_op, acc_dev, didx, ebuf, sem):
        c = lax.axis_index("c")
        s = lax.axis_index("s")
        w = c * _NT + s

        def scatter_etype(ef_h, acc, d_h, rows):
            nb = rows // (2 * _NT * 8)

            def body(bi, carry):
                rbase = (w * nb + bi) * 8
                pltpu.sync_copy(d_h.at[pl.ds(rbase, 8)], didx)
                hs = [pltpu.async_copy(
                    ef_h.at[pl.ds((rbase + j) * 128, 128)], ebuf.at[j], sem)
                    for j in range(8)]
                for j in range(8):
                    hs[j].wait()
                for j in range(8):
                    pltpu.sync_copy(ebuf.at[j], acc.at[didx.at[j]], add=True)
                return carry

            lax.fori_loop(0, nb, body, 0)

        nop_pt = _NOPP // _NT
        ndev_pt = _NDEVP // _NT

        def zero(acc, npt):
            pltpu.sync_copy(z16_h.at[pl.ds(0, npt)],
                            acc.at[pl.ds(s * npt, npt)])

        def evac(acc, e_h, npt, accrows):
            pltpu.sync_copy(acc.at[pl.ds(s * npt, npt)],
                            e_h.at[pl.ds(c * accrows + s * npt, npt)])

        zero(acc_op, nop_pt)
        zero(acc_dev, ndev_pt)
        plsc.subcore_barrier()
        scatter_etype(ef1_h, acc_op, d1_h, d1_h.shape[0])
        plsc.subcore_barrier()
        evac(acc_op, e1_h, nop_pt, _NOPP)
        zero(acc_op, nop_pt)
        plsc.subcore_barrier()
        scatter_etype(ef2_h, acc_op, d2_h, d2_h.shape[0])
        plsc.subcore_barrier()
        evac(acc_op, e2_h, nop_pt, _NOPP)
        zero(acc_op, nop_pt)
        plsc.subcore_barrier()
        scatter_etype(ef4_h, acc_op, d4_h, d4_h.shape[0])
        scatter_etype(ef0_h, acc_dev, d0_h, d0_h.shape[0])
        plsc.subcore_barrier()
        evac(acc_op, e4_h, nop_pt, _NOPP)
        evac(acc_dev, e0_h, ndev_pt, _NDEVP)
        zero(acc_dev, ndev_pt)
        plsc.subcore_barrier()
        scatter_etype(ef3_h, acc_dev, d3_h, d3_h.shape[0])
        plsc.subcore_barrier()
        evac(acc_dev, e3_h, ndev_pt, _NDEVP)

    es0, es1, es2, es3, es4 = k(efs[0], efs[1], efs[2], efs[3], efs[4],
                                didxs[0], didxs[1], didxs[2], didxs[3],
                                didxs[4], z16)
    return [es0.reshape(2, _NDEVP, 16), es1.reshape(2, _NOPP, 16),
            es2.reshape(2, _NOPP, 16), es3.reshape(2, _NDEVP, 16),
            es4.reshape(2, _NOPP, 16)]


def _sc_final_gather(a_tab, b_tab, ps_rows, pd_rows):
    rp = ps_rows.shape[0]
    out_sds = jax.ShapeDtypeStruct((rp * 128, 16), jnp.float32)

    @functools.partial(
        pl.kernel,
        mesh=plsc.VectorSubcoreMesh(**_SC_MESH),
        compiler_params=pltpu.CompilerParams(use_tc_tiling_on_sc=False),
        out_type=(out_sds, out_sds),
        scratch_types=[
            pltpu.VMEM((8, 128), jnp.int32),
            pltpu.VMEM((8, 128, 16), jnp.float32),
            pltpu.SemaphoreType.DMA,
        ],
    )
    def k(a_h, b_h, ps_h, pd_h, ga_h, gb_h, idx, buf, sem):
        c = lax.axis_index("c")
        s = lax.axis_index("s")
        w = c * _NT + s
        nb = rp // (2 * _NT * 8)

        def gather_tab(tab_h, i_h, o_h):
            def body(bi, carry):
                rbase = (w * nb + bi) * 8
                pltpu.sync_copy(i_h.at[pl.ds(rbase, 8)], idx)
                hs = [pltpu.async_copy(tab_h.at[idx.at[j]], buf.at[j], sem)
                      for j in range(8)]
                for j in range(8):
                    hs[j].wait()
                for j in range(8):
                    pltpu.sync_copy(buf.at[j],
                                    o_h.at[pl.ds((rbase + j) * 128, 128)])
                return carry

            lax.fori_loop(0, nb, body, 0)

        gather_tab(a_h, ps_h, ga_h)
        gather_tab(b_h, pd_h, gb_h)

    return k(a_tab, b_tab, ps_rows, pd_rows)


def kernel(op_feats, device_feats, tensor_feats, link_feats, place_feats,
           prev_edge_index, link_edge_index, place_edge_index,
           op_W, op_b, dev_W, dev_b, et_W, et_b, gconv_W, gconv_b,
           fp_W, fp_b, fn_W, fn_b):
    e_p = place_feats.shape[0]

    def pad_rows(x, m=_EPM):
        r = (-x.shape[0]) % m
        return x if r == 0 else jnp.pad(x, ((0, r), (0, 0)))

    def pad_idx(idx, pad_val):
        e = idx.shape[0]
        ep = -(-e // _EPM) * _EPM
        a = jnp.full((ep,), pad_val, jnp.int32).at[:e].set(idx)
        return a.reshape(ep // 128, 128)

    opf = _init_nodes(op_feats, op_W, op_b, bm=2000)
    devf = _init_nodes(device_feats, dev_W, dev_b, bm=1024)

    ef = [None] * 5
    ef[0] = _edge_mlp(pad_rows(link_feats), et_W[0], et_b[0], bm=4096)
    ef[1] = _edge_mlp(pad_rows(tensor_feats), et_W[1], et_b[1], bm=4096)
    ef[2] = _edge_mlp(pad_rows(tensor_feats), et_W[2], et_b[2], bm=4096)
    ef[3] = _edge_mlp(pad_rows(place_feats), et_W[3], et_b[3], bm=4096)
    ef[4] = _edge_mlp(pad_rows(place_feats), et_W[4], et_b[4], bm=4096)

    src_idx = [link_edge_index[0], prev_edge_index[0], prev_edge_index[1],
               place_edge_index[0], place_edge_index[1]]
    dst_idx = [link_edge_index[1], prev_edge_index[1], prev_edge_index[0],
               place_edge_index[1], place_edge_index[0]]
    src_is_op = [False, True, True, True, False]
    dst_is_op = [False, True, True, False, True]

    idxs = []
    didxs = []
    for i in range(5):
        s_off = _NOP if src_is_op[i] else _NDEV
        d_dum = _NOP if dst_is_op[i] else _NDEV
        srows = pad_idx(src_idx[i], 0)
        gidx = jnp.concatenate([srows, srows + s_off], axis=0)
        didx = pad_idx(dst_idx[i], d_dum)
        idxs.append((gidx, didx))
        didxs.append(didx)

    z32 = jnp.zeros((_NOPP // _NT, 32), jnp.float32)
    z16 = jnp.zeros((_NOPP // _NT, 16), jnp.float32)

    es = _sc_es_kernel(ef, didxs, z16)

    for l in range(_NL):
        gs = _sc_g_kernel(opf.reshape(2 * _NOP, 32),
                          devf.reshape(2 * _NDEV, 32), z32, idxs)
        last = (l == _NL - 1)
        op_ks = [1, 2, 4]
        dev_ks = [0, 3]
        opf_n = _combine(opf, [gs[k] for k in op_ks], [es[k] for k in op_ks],
                         [gconv_W[l, k, :_H] for k in op_ks],
                         [gconv_W[l, k, _H:] for k in op_ks],
                         [gconv_b[l, k] for k in op_ks], last, bm=2000)
        devf_n = _combine(devf, [gs[k] for k in dev_ks], [es[k] for k in dev_ks],
                          [gconv_W[l, k, :_H] for k in dev_ks],
                          [gconv_W[l, k, _H:] for k in dev_ks],
                          [gconv_b[l, k] for k in dev_ks], last, bm=1024)
        opf, devf = opf_n, devf_n

    wa = jnp.zeros((64, 16), jnp.float32)
    wa = wa.at[:, :3].set(fp_W[:_H])
    wa = wa.at[:, 3:4].set(fn_W)
    ba = jnp.zeros((16,), jnp.float32).at[3].set(fn_b[0])
    a_tab = _proj_head(opf, wa, ba, bm=2000)
    wb = jnp.zeros((64, 16), jnp.float32)
    wb = wb.at[:, :3].set(fp_W[_H + _EH:])
    b_tab = _proj_head(devf, wb, jnp.zeros((16,), jnp.float32), bm=1024)

    ga, gb = _sc_final_gather(a_tab, b_tab,
                              pad_idx(place_edge_index[0], 0),
                              pad_idx(place_edge_index[1], 0))

    wfin = jnp.zeros((16, 16), jnp.float32)
    wfin = wfin.at[:_EH, :3].set(fp_W[_H:_H + _EH])
    wfin = wfin.at[_EH, :3].set(fp_b)
    d16 = _final_edge(ga, gb, ef[3], wfin, bm=4096)
    d = d16[:e_p, :3]
    nccl = a_tab[:, 3]
    return (d, nccl)


# speedup vs baseline: 3.1417x; 1.0109x over previous
import functools
import jax
import jax.numpy as jnp
from jax import lax
from jax.experimental import pallas as pl
from jax.experimental.pallas import tpu as pltpu
from jax.experimental.pallas import tpu_sc as plsc

_NOP = 50000
_NDEV = 1024
_H = 64
_EH = 8
_NL = 6

_NC = 2
_NT = 16
_NOPP = _NOP + 48
_NDEVP = _NDEV + 128
_EPM = 2 * _NT * 8 * 128


def _elu(x):
    return jnp.where(x > 0, x, jnp.exp(jnp.minimum(x, 0.0)) - 1.0)


def _init_nodes_body(x_ref, w_ref, b_ref, o_ref):
    y = jnp.dot(x_ref[...], w_ref[0], preferred_element_type=jnp.float32)
    o_ref[0] = _elu(y + b_ref[0])


def _init_nodes(x, w, b, bm):
    n, din = x.shape
    nb = n // bm
    wst = w.reshape(din, 2, 32).transpose(1, 0, 2)
    bst = b.reshape(2, 1, 32)
    return pl.pallas_call(
        _init_nodes_body,
        grid=(2, nb),
        in_specs=[
            pl.BlockSpec((bm, din), lambda h, i: (i, 0)),
            pl.BlockSpec((1, din, 32), lambda h, i: (h, 0, 0)),
            pl.BlockSpec((1, 1, 32), lambda h, i: (h, 0, 0)),
        ],
        out_specs=pl.BlockSpec((1, bm, 32), lambda h, i: (h, i, 0)),
        out_shape=jax.ShapeDtypeStruct((2, n, 32), jnp.float32),
    )(x, wst, bst)


def _edge_mlp_body(x_ref, w_ref, b_ref, o_ref):
    z = jnp.dot(x_ref[...], w_ref[...], preferred_element_type=jnp.float32)
    z = z + b_ref[0]
    col = lax.broadcasted_iota(jnp.int32, z.shape, 1)
    o_ref[...] = jnp.where(col < _EH, _elu(z),
                           jnp.where(col == _EH, 1.0, 0.0))


def _edge_mlp(raw_pad, w, b, bm):
    ep = raw_pad.shape[0]
    w16 = jnp.pad(w, ((0, 0), (0, 16 - _EH)))
    b16 = jnp.pad(b, (0, 16 - _EH)).reshape(1, 16)
    return pl.pallas_call(
        _edge_mlp_body,
        grid=(ep // bm,),
        in_specs=[
            pl.BlockSpec((bm, 16), lambda i: (i, 0)),
            pl.BlockSpec((16, 16), lambda i: (0, 0)),
            pl.BlockSpec((1, 16), lambda i: (0, 0)),
        ],
        out_specs=pl.BlockSpec((bm, 16), lambda i: (i, 0)),
        out_shape=jax.ShapeDtypeStruct((ep, 16), jnp.float32),
    )(raw_pad, w16, b16)


def _combine_body(nark, op_ref,
                  g0l, g0h, g1l, g1h, g2l, g2h,
                  e0a, e0b, e1a, e1b, e2a, e2b,
                  w0, w1, w2, we0, we1, we2, o_ref,
                  *, last):
    gl = (g0l, g1l, g2l)
    gh = (g0h, g1h, g2h)
    ea = (e0a, e1a, e2a)
    eb = (e0b, e1b, e2b)
    ws = (w0, w1, w2)
    wes = (we0, we1, we2)
    acc = jnp.zeros(op_ref.shape[1:], jnp.float32)
    for k in range(nark):
        es = ea[k][0] + eb[k][0]
        cnt = es[:, _EH:_EH + 1]
        num = jnp.dot(gl[k][0], ws[k][0, 0], preferred_element_type=jnp.float32)
        num += jnp.dot(gh[k][0], ws[k][0, 1], preferred_element_type=jnp.float32)
        num += jnp.dot(es, wes[k][0], preferred_element_type=jnp.float32)
        acc += num / jnp.maximum(cnt, 1.0)
    o = op_ref[0] + acc * (1.0 / nark)
    o_ref[0] = o if last else _elu(o)


def _combine(opf, gs, ess, whs, wes, bs, last, bm):
    nark = len(gs)
    n = opf.shape[1]
    nb = n // bm
    wsts, wests = [], []
    for k in range(nark):
        w = whs[k].reshape(2, 32, 2, 32).transpose(2, 0, 1, 3)
        wsts.append(w)
        wep = jnp.zeros((16, 64), jnp.float32)
        wep = wep.at[:_EH].set(wes[k])
        wep = wep.at[_EH].set(bs[k])
        wests.append(wep.reshape(16, 2, 32).transpose(1, 0, 2))
    big = pl.BlockSpec((1, bm, 32), lambda h, i: (0, i, 0))
    bigh = pl.BlockSpec((1, bm, 32), lambda h, i: (1, i, 0))
    esa = pl.BlockSpec((1, bm, 16), lambda h, i: (0, i, 0))
    esb = pl.BlockSpec((1, bm, 16), lambda h, i: (1, i, 0))
    wsp = pl.BlockSpec((1, 2, 32, 32), lambda h, i: (h, 0, 0, 0))
    wesp = pl.BlockSpec((1, 16, 32), lambda h, i: (h, 0, 0))
    g3 = [gs[k] if k < nark else gs[0] for k in range(3)]
    e3 = [ess[k] if k < nark else ess[0] for k in range(3)]
    w3 = [wsts[k] if k < nark else wsts[0] for k in range(3)]
    we3 = [wests[k] if k < nark else wests[0] for k in range(3)]
    in_specs = [pl.BlockSpec((1, bm, 32), lambda h, i: (h, i, 0))]
    args = [opf]
    for k in range(3):
        in_specs += [big, bigh]
        args += [g3[k], g3[k]]
    for k in range(3):
        in_specs += [esa, esb]
        args += [e3[k], e3[k]]
    in_specs += [wsp] * 3 + [wesp] * 3
    args += w3 + we3
    return pl.pallas_call(
        functools.partial(_combine_body, len(gs), last=last),
        grid=(2, nb),
        in_specs=in_specs,
        out_specs=pl.BlockSpec((1, bm, 32), lambda h, i: (h, i, 0)),
        out_shape=jax.ShapeDtypeStruct((2, n, 32), jnp.float32),
    )(*args)


def _proj_head_body(xl_ref, xh_ref, w_ref, b_ref, o_ref):
    y = jnp.dot(xl_ref[0], w_ref[:32], preferred_element_type=jnp.float32)
    y += jnp.dot(xh_ref[0], w_ref[32:], preferred_element_type=jnp.float32)
    o_ref[...] = y + b_ref[...]


def _proj_head(opf, w64x16, b16, bm):
    n = opf.shape[1]
    return pl.pallas_call(
        _proj_head_body,
        grid=(n // bm,),
        in_specs=[
            pl.BlockSpec((1, bm, 32), lambda i: (0, i, 0)),
            pl.BlockSpec((1, bm, 32), lambda i: (1, i, 0)),
            pl.BlockSpec((64, 16), lambda i: (0, 0)),
            pl.BlockSpec((1, 16), lambda i: (0, 0)),
        ],
        out_specs=pl.BlockSpec((bm, 16), lambda i: (i, 0)),
        out_shape=jax.ShapeDtypeStruct((n, 16), jnp.float32),
    )(opf, opf, w64x16, b16.reshape(1, 16))


def _final_edge_body(ga_ref, gb_ref, ef_ref, w_ref, o_ref):
    y = ga_ref[...] + gb_ref[...]
    y += jnp.dot(ef_ref[...], w_ref[...], preferred_element_type=jnp.float32)
    o_ref[...] = y


def _final_edge(ga, gb, ef3, wfin, bm):
    ep = ga.shape[0]
    return pl.pallas_call(
        _final_edge_body,
        grid=(ep // bm,),
        in_specs=[
            pl.BlockSpec((bm, 16), lambda i: (i, 0)),
            pl.BlockSpec((bm, 16), lambda i: (i, 0)),
            pl.BlockSpec((bm, 16), lambda i: (i, 0)),
            pl.BlockSpec((16, 16), lambda i: (0, 0)),
        ],
        out_specs=pl.BlockSpec((bm, 16), lambda i: (i, 0)),
        out_shape=jax.ShapeDtypeStruct((ep, 16), jnp.float32),
    )(ga, gb, ef3, wfin)


_SC_MESH = dict(core_axis_name="c", subcore_axis_name="s",
                num_cores=_NC, num_subcores=_NT)


def _sc_g_kernel(opf2, devf2, z32, idxs):
    rt = idxs[1][1].shape[0]
    rp = idxs[3][1].shape[0]
    rl = idxs[0][1].shape[0]
    op_sds = jax.ShapeDtypeStruct((2 * _NOPP, 32), jnp.float32)
    dev_sds = jax.ShapeDtypeStruct((2 * _NDEVP, 32), jnp.float32)

    @functools.partial(
        pl.kernel,
        mesh=plsc.VectorSubcoreMesh(**_SC_MESH),
        compiler_params=pltpu.CompilerParams(use_tc_tiling_on_sc=False),
        out_type=(op_sds, op_sds, op_sds, dev_sds, dev_sds),
        scratch_types=[
            pltpu.VMEM_SHARED((_NOPP, 32), jnp.float32),
            pltpu.VMEM_SHARED((_NDEVP, 32), jnp.float32),
            pltpu.VMEM((8, 128), jnp.int32),
            pltpu.VMEM((8, 128), jnp.int32),
            pltpu.VMEM((6, 128, 32), jnp.float32),
            pltpu.SemaphoreType.DMA,
            pltpu.SemaphoreType.DMA,
            pltpu.SemaphoreType.DMA,
            pltpu.SemaphoreType.DMA,
            pltpu.SemaphoreType.DMA,
            pltpu.SemaphoreType.DMA,
        ],
    )
    def k(opf2_h, devf2_h, z32_h,
          s1_h, d1_h, s2_h, d2_h, s4_h, d4_h, s0_h, d0_h, s3_h, d3_h,
          g1_h, g2_h, g4_h, g0_h, g3_h,
          acc_op, acc_dev, sidx, didx, gbuf, sm0, sm1, sm2, sm3, sm4, sm5):
        sems = (sm0, sm1, sm2, sm3, sm4, sm5)
        c = lax.axis_index("c")
        s = lax.axis_index("s")

        def scatter_etype(tab_h, acc, s_h, d_h, rows):
            nb = rows // (_NT * 8)

            def body(bi, carry):
                rbase = (s * nb + bi) * 8
                pltpu.sync_copy(s_h.at[pl.ds(c * rows + rbase, 8)], sidx)
                pltpu.sync_copy(d_h.at[pl.ds(rbase, 8)], didx)

                def fire_g(j):
                    return pltpu.async_copy(tab_h.at[sidx.at[j]],
                                            gbuf.at[j % 6], sems[j % 6])

                def fire_s(j):
                    return pltpu.async_copy(gbuf.at[j % 6],
                                            acc.at[didx.at[j]],
                                            sems[j % 6], add=True)

                hg = {j: fire_g(j) for j in range(6)}
                hs = {}
                for j in range(8):
                    hg[j].wait()
                    hs[j] = fire_s(j)
                    if 2 <= j <= 3:
                        hs[j - 2].wait()
                        hg[j + 4] = fire_g(j + 4)
                for j in range(2, 8):
                    hs[j].wait()
                return carry

            lax.fori_loop(0, nb, body, 0)

        nop_pt = _NOPP // _NT
        ndev_pt = _NDEVP // _NT

        def zero(acc, npt):
            pltpu.sync_copy(z32_h.at[pl.ds(0, npt)],
                            acc.at[pl.ds(s * npt, npt)])

        def evac(acc, g_h, npt, accrows):
            pltpu.sync_copy(acc.at[pl.ds(s * npt, npt)],
                            g_h.at[pl.ds(c * accrows + s * npt, npt)])

        zero(acc_op, nop_pt)
        zero(acc_dev, ndev_pt)
        plsc.subcore_barrier()
        scatter_etype(opf2_h, acc_op, s1_h, d1_h, rt)
        plsc.subcore_barrier()
        evac(acc_op, g1_h, nop_pt, _NOPP)
        zero(acc_op, nop_pt)
        plsc.subcore_barrier()
        scatter_etype(opf2_h, acc_op, s2_h, d2_h, rt)
        plsc.subcore_barrier()
        evac(acc_op, g2_h, nop_pt, _NOPP)
        zero(acc_op, nop_pt)
        plsc.subcore_barrier()
        scatter_etype(devf2_h, acc_op, s4_h, d4_h, rp)
        scatter_etype(devf2_h, acc_dev, s0_h, d0_h, rl)
        plsc.subcore_barrier()
        evac(acc_op, g4_h, nop_pt, _NOPP)
        evac(acc_dev, g0_h, ndev_pt, _NDEVP)
        zero(acc_dev, ndev_pt)
        plsc.subcore_barrier()
        scatter_etype(opf2_h, acc_dev, s3_h, d3_h, rp)
        plsc.subcore_barrier()
        evac(acc_dev, g3_h, ndev_pt, _NDEVP)

    g1, g2, g4, g0, g3 = k(
        opf2, devf2, z32,
        idxs[1][0], idxs[1][1], idxs[2][0], idxs[2][1],
        idxs[4][0], idxs[4][1], idxs[0][0], idxs[0][1],
        idxs[3][0], idxs[3][1])
    return [g0.reshape(2, _NDEVP, 32), g1.reshape(2, _NOPP, 32),
            g2.reshape(2, _NOPP, 32), g3.reshape(2, _NDEVP, 32),
            g4.reshape(2, _NOPP, 32)]


def _sc_es_kernel(efs, didxs, z16):
    op_sds = jax.ShapeDtypeStruct((2 * _NOPP, 16), jnp.float32)
    dev_sds = jax.ShapeDtypeStruct((2 * _NDEVP, 16), jnp.float32)

    @functools.partial(
        pl.kernel,
        mesh=plsc.VectorSubcoreMesh(**_SC_MESH),
        compiler_params=pltpu.CompilerParams(use_tc_tiling_on_sc=False),
        out_type=(dev_sds, op_sds, op_sds, dev_sds, op_sds),
        scratch_types=[
            pltpu.VMEM_SHARED((_NOPP, 16), jnp.float32),
            pltpu.VMEM_SHARED((_NDEVP, 16), jnp.float32),
            pltpu.VMEM((8, 128), jnp.int32),
            pltpu.VMEM((8, 128, 16), jnp.float32),
            pltpu.SemaphoreType.DMA,
        ],
    )
    def k(ef0_h, ef1_h, ef2_h, ef3_h, ef4_h,
          d0_h, d1_h, d2_h, d3_h, d4_h, z16_h,
          e0_h, e1_h, e2_h, e3_h, e4_h,
          acc_op, acc_dev, didx, ebuf, sem):
        c = lax.axis_index("c")
        s = lax.axis_index("s")
        w = c * _NT + s

        def scatter_etype(ef_h, acc, d_h, rows):
            nb = rows // (2 * _NT * 8)

            def body(bi, carry):
                rbase = (w * nb + bi) * 8
                pltpu.sync_copy(d_h.at[pl.ds(rbase, 8)], didx)
                hs = [pltpu.async_copy(
                    ef_h.at[pl.ds((rbase + j) * 128, 128)], ebuf.at[j], sem)
                    for j in range(8)]
                for j in range(8):
                    hs[j].wait()
                for j in range(8):
                    pltpu.sync_copy(ebuf.at[j], acc.at[didx.at[j]], add=True)
                return carry

            lax.fori_loop(0, nb, body, 0)

        nop_pt = _NOPP // _NT
        ndev_pt = _NDEVP // _NT

        def zero(acc, npt):
            pltpu.sync_copy(z16_h.at[pl.ds(0, npt)],
                            acc.at[pl.ds(s * npt, npt)])

        def evac(acc, e_h, npt, accrows):
            pltpu.sync_copy(acc.at[pl.ds(s * npt, npt)],
                            e_h.at[pl.ds(c * accrows + s * npt, npt)])

        zero(acc_op, nop_pt)
        zero(acc_dev, ndev_pt)
        plsc.subcore_barrier()
        scatter_etype(ef1_h, acc_op, d1_h, d1_h.shape[0])
        plsc.subcore_barrier()
        evac(acc_op, e1_h, nop_pt, _NOPP)
        zero(acc_op, nop_pt)
        plsc.subcore_barrier()
        scatter_etype(ef2_h, acc_op, d2_h, d2_h.shape[0])
        plsc.subcore_barrier()
        evac(acc_op, e2_h, nop_pt, _NOPP)
        zero(acc_op, nop_pt)
        plsc.subcore_barrier()
        scatter_etype(ef4_h, acc_op, d4_h, d4_h.shape[0])
        scatter_etype(ef0_h, acc_dev, d0_h, d0_h.shape[0])
        plsc.subcore_barrier()
        evac(acc_op, e4_h, nop_pt, _NOPP)
        evac(acc_dev, e0_h, ndev_pt, _NDEVP)
        zero(acc_dev, ndev_pt)
        plsc.subcore_barrier()
        scatter_etype(ef3_h, acc_dev, d3_h, d3_h.shape[0])
        plsc.subcore_barrier()
        evac(acc_dev, e3_h, ndev_pt, _NDEVP)

    es0, es1, es2, es3, es4 = k(efs[0], efs[1], efs[2], efs[3], efs[4],
                                didxs[0], didxs[1], didxs[2], didxs[3],
                                didxs[4], z16)
    return [es0.reshape(2, _NDEVP, 16), es1.reshape(2, _NOPP, 16),
            es2.reshape(2, _NOPP, 16), es3.reshape(2, _NDEVP, 16),
            es4.reshape(2, _NOPP, 16)]


def _sc_final_gather(a_tab, b_tab, ps_rows, pd_rows):
    rp = ps_rows.shape[0]
    out_sds = jax.ShapeDtypeStruct((rp * 128, 16), jnp.float32)

    @functools.partial(
        pl.kernel,
        mesh=plsc.VectorSubcoreMesh(**_SC_MESH),
        compiler_params=pltpu.CompilerParams(use_tc_tiling_on_sc=False),
        out_type=(out_sds, out_sds),
        scratch_types=[
            pltpu.VMEM((8, 128), jnp.int32),
            pltpu.VMEM((8, 128, 16), jnp.float32),
            pltpu.SemaphoreType.DMA,
        ],
    )
    def k(a_h, b_h, ps_h, pd_h, ga_h, gb_h, idx, buf, sem):
        c = lax.axis_index("c")
        s = lax.axis_index("s")
        w = c * _NT + s
        nb = rp // (2 * _NT * 8)

        def gather_tab(tab_h, i_h, o_h):
            def body(bi, carry):
                rbase = (w * nb + bi) * 8
                pltpu.sync_copy(i_h.at[pl.ds(rbase, 8)], idx)
                hs = [pltpu.async_copy(tab_h.at[idx.at[j]], buf.at[j], sem)
                      for j in range(8)]
                for j in range(8):
                    hs[j].wait()
                for j in range(8):
                    pltpu.sync_copy(buf.at[j],
                                    o_h.at[pl.ds((rbase + j) * 128, 128)])
                return carry

            lax.fori_loop(0, nb, body, 0)

        gather_tab(a_h, ps_h, ga_h)
        gather_tab(b_h, pd_h, gb_h)

    return k(a_tab, b_tab, ps_rows, pd_rows)


def kernel(op_feats, device_feats, tensor_feats, link_feats, place_feats,
           prev_edge_index, link_edge_index, place_edge_index,
           op_W, op_b, dev_W, dev_b, et_W, et_b, gconv_W, gconv_b,
           fp_W, fp_b, fn_W, fn_b):
    e_p = place_feats.shape[0]

    def pad_rows(x, m=_EPM):
        r = (-x.shape[0]) % m
        return x if r == 0 else jnp.pad(x, ((0, r), (0, 0)))

    def pad_idx(idx, pad_val):
        e = idx.shape[0]
        ep = -(-e // _EPM) * _EPM
        a = jnp.full((ep,), pad_val, jnp.int32).at[:e].set(idx)
        return a.reshape(ep // 128, 128)

    opf = _init_nodes(op_feats, op_W, op_b, bm=2000)
    devf = _init_nodes(device_feats, dev_W, dev_b, bm=1024)

    ef = [None] * 5
    ef[0] = _edge_mlp(pad_rows(link_feats), et_W[0], et_b[0], bm=4096)
    ef[1] = _edge_mlp(pad_rows(tensor_feats), et_W[1], et_b[1], bm=4096)
    ef[2] = _edge_mlp(pad_rows(tensor_feats), et_W[2], et_b[2], bm=4096)
    ef[3] = _edge_mlp(pad_rows(place_feats), et_W[3], et_b[3], bm=4096)
    ef[4] = _edge_mlp(pad_rows(place_feats), et_W[4], et_b[4], bm=4096)

    src_idx = [link_edge_index[0], prev_edge_index[0], prev_edge_index[1],
               place_edge_index[0], place_edge_index[1]]
    dst_idx = [link_edge_index[1], prev_edge_index[1], prev_edge_index[0],
               place_edge_index[1], place_edge_index[0]]
    src_is_op = [False, True, True, True, False]
    dst_is_op = [False, True, True, False, True]

    idxs = []
    didxs = []
    for i in range(5):
        s_off = _NOP if src_is_op[i] else _NDEV
        d_dum = _NOP if dst_is_op[i] else _NDEV
        srows = pad_idx(src_idx[i], 0)
        gidx = jnp.concatenate([srows, srows + s_off], axis=0)
        didx = pad_idx(dst_idx[i], d_dum)
        idxs.append((gidx, didx))
        didxs.append(didx)

    z32 = jnp.zeros((_NOPP // _NT, 32), jnp.float32)
    z16 = jnp.zeros((_NOPP // _NT, 16), jnp.float32)

    es = _sc_es_kernel(ef, didxs, z16)

    for l in range(_NL):
        gs = _sc_g_kernel(opf.reshape(2 * _NOP, 32),
                          devf.reshape(2 * _NDEV, 32), z32, idxs)
        last = (l == _NL - 1)
        op_ks = [1, 2, 4]
        dev_ks = [0, 3]
        opf_n = _combine(opf, [gs[k] for k in op_ks], [es[k] for k in op_ks],
                         [gconv_W[l, k, :_H] for k in op_ks],
                         [gconv_W[l, k, _H:] for k in op_ks],
                         [gconv_b[l, k] for k in op_ks], last, bm=2000)
        devf_n = _combine(devf, [gs[k] for k in dev_ks], [es[k] for k in dev_ks],
                          [gconv_W[l, k, :_H] for k in dev_ks],
                          [gconv_W[l, k, _H:] for k in dev_ks],
                          [gconv_b[l, k] for k in dev_ks], last, bm=1024)
        opf, devf = opf_n, devf_n

    wa = jnp.zeros((64, 16), jnp.float32)
    wa = wa.at[:, :3].set(fp_W[:_H])
    wa = wa.at[:, 3:4].set(fn_W)
    ba = jnp.zeros((16,), jnp.float32).at[3].set(fn_b[0])
    a_tab = _proj_head(opf, wa, ba, bm=2000)
    wb = jnp.zeros((64, 16), jnp.float32)
    wb = wb.at[:, :3].set(fp_W[_H + _EH:])
    b_tab = _proj_head(devf, wb, jnp.zeros((16,), jnp.float32), bm=1024)

    ga, gb = _sc_final_gather(a_tab, b_tab,
                              pad_idx(place_edge_index[0], 0),
                              pad_idx(place_edge_index[1], 0))

    wfin = jnp.zeros((16, 16), jnp.float32)
    wfin = wfin.at[:_EH, :3].set(fp_W[_H:_H + _EH])
    wfin = wfin.at[_EH, :3].set(fp_b)
    d16 = _final_edge(ga, gb, ef[3], wfin, bm=4096)
    d = d16[:e_p, :3]
    nccl = a_tab[:, 3]
    return (d, nccl)
